# baseline (speedup 1.0000x reference)
import functools

import jax
import jax.numpy as jnp
from jax import lax
from jax.experimental import pallas as pl
from jax.experimental.pallas import tpu as pltpu

_MXU_DTYPE = jnp.bfloat16


def _fused_kernel(x_ref, lens_ref, wih_ref, bcat_ref, whhbd_ref,
                  w1_ref, b1_ref, w2_ref, b2_ref, w3_ref, b3_ref,
                  out_ref,
                  xbuf, xsem):
    T, Bt, E = x_ref.shape
    H = whhbd_ref.shape[0] // 2
    G4 = 4 * H

    def xcopy(s, t, d):
        return pltpu.make_async_copy(x_ref.at[t], xbuf.at[s % 2, d],
                                     xsem.at[s % 2, d])

    xcopy(0, 0, 0).start()
    xcopy(0, T - 1, 1).start()

    bdt = whhbd_ref.dtype
    wih_f = wih_ref[:, :G4]
    wih_b = wih_ref[:, G4:]
    b_f = bcat_ref[:, :G4]
    b_b = bcat_ref[:, G4:]
    whh_f = whhbd_ref[:H]
    whh_b = whhbd_ref[H:]
    lens = lens_ref[...]

    def gate_math(g, c_prev):
        i = 0.5 * jnp.tanh(g[:, 0 * H:1 * H]) + 0.5
        f = 0.5 * jnp.tanh(g[:, 1 * H:2 * H]) + 0.5
        gg = jnp.tanh(g[:, 2 * H:3 * H])
        o = 0.5 * jnp.tanh(g[:, 3 * H:4 * H]) + 0.5
        c_new = f * c_prev + i * gg
        h_new = o * jnp.tanh(c_new)
        return h_new, c_new

    zeros_bf = jnp.zeros((Bt, H), bdt)
    zeros = jnp.zeros((Bt, H), jnp.float32)
    neg = jnp.full((Bt, H), -jnp.inf, jnp.float32)
    h_f, c_f, m_f, h_b, c_b, m_b = (
        zeros_bf, zeros, neg, zeros_bf, zeros, neg)

    for s in range(T):
        tb = T - 1 - s
        if s + 1 < T:
            xcopy(s + 1, s + 1, 0).start()
            xcopy(s + 1, tb - 1, 1).start()
        xcopy(s, s, 0).wait()
        xcopy(s, tb, 1).wait()
        gf = (jnp.dot(xbuf[s % 2, 0].astype(bdt), wih_f,
                      preferred_element_type=jnp.float32)
              + jnp.dot(h_f, whh_f, preferred_element_type=jnp.float32)
              + b_f)
        gb = (jnp.dot(xbuf[s % 2, 1].astype(bdt), wih_b,
                      preferred_element_type=jnp.float32)
              + jnp.dot(h_b, whh_b, preferred_element_type=jnp.float32)
              + b_b)
        vf = s < lens
        vb = tb < lens
        hf_new, cf_new = gate_math(gf, c_f)
        hb_new, cb_new = gate_math(gb, c_b)
        m_f = jnp.maximum(m_f, jnp.where(vf, hf_new, 0.0))
        h_b = jnp.where(vb, hb_new.astype(bdt), h_b)
        c_b = jnp.where(vb, cb_new, c_b)
        m_b = jnp.maximum(m_b, jnp.where(vb, hb_new, 0.0))
        h_f, c_f = hf_new.astype(bdt), cf_new

    pooled = jnp.concatenate([m_f, m_b], axis=1)
    P = Bt // 2
    u = pooled[:P]
    v = pooled[P:]
    wdt = w1_ref.dtype
    feats = jnp.concatenate([u, v, jnp.abs(u - v), u * v],
                            axis=1).astype(wdt)
    h1 = (jnp.dot(feats, w1_ref[...], preferred_element_type=jnp.float32)
          + b1_ref[...])
    h2 = (jnp.dot(h1.astype(wdt), w2_ref[...],
                  preferred_element_type=jnp.float32) + b2_ref[...])
    out_ref[...] = (jnp.dot(h2.astype(wdt), w3_ref[...],
                            preferred_element_type=jnp.float32) + b3_ref[...])


def kernel(sentence1, lengths1, sentence2, lengths2, embedding,
           wih_cat_t, b_cat, whh_f_t, whh_b_t,
           w1_u, w1_v, w1_d, w1_p, b1, w2_t, b2, w3_t, b3):
    B, T = sentence1.shape
    E = embedding.shape[1]
    H = whh_f_t.shape[0]
    L = w3_t.shape[1]
    P = 128
    assert B % P == 0
    ntiles = B // P
    Bt = 2 * P

    gate_scale = jnp.concatenate(
        [jnp.full((1, H), 0.5, jnp.float32),
         jnp.full((1, H), 0.5, jnp.float32),
         jnp.ones((1, H), jnp.float32),
         jnp.full((1, H), 0.5, jnp.float32)], axis=1)
    gs2 = jnp.concatenate([gate_scale, gate_scale], axis=1)
    wih_s = wih_cat_t * gs2
    bcat_s = b_cat * gs2
    whh_bd = (jnp.concatenate([whh_f_t, whh_b_t], axis=0)
              * gate_scale).astype(_MXU_DTYPE)
    w1_full = jnp.concatenate([w1_u, w1_v, w1_d, w1_p],
                              axis=0).astype(_MXU_DTYPE)

    grid_spec = pltpu.PrefetchScalarGridSpec(
        num_scalar_prefetch=0,
        grid=(1,),
        in_specs=[
            pl.BlockSpec(memory_space=pltpu.MemorySpace.HBM),
            pl.BlockSpec((Bt, 1), lambda i: (0, 0)),
            pl.BlockSpec((E, 8 * H), lambda i: (0, 0)),
            pl.BlockSpec((1, 8 * H), lambda i: (0, 0)),
            pl.BlockSpec((2 * H, 4 * H), lambda i: (0, 0)),
            pl.BlockSpec((8 * H, w1_full.shape[1]), lambda i: (0, 0)),
            pl.BlockSpec((1, b1.shape[1]), lambda i: (0, 0)),
            pl.BlockSpec(w2_t.shape, lambda i: (0, 0)),
            pl.BlockSpec((1, b2.shape[1]), lambda i: (0, 0)),
            pl.BlockSpec(w3_t.shape, lambda i: (0, 0)),
            pl.BlockSpec((1, b3.shape[1]), lambda i: (0, 0)),
        ],
        out_specs=pl.BlockSpec((P, L), lambda i: (0, 0)),
        scratch_shapes=[pltpu.VMEM((2, 2, Bt, E), jnp.float32),
                        pltpu.SemaphoreType.DMA((2, 2))],
    )
    call = pl.pallas_call(
        _fused_kernel,
        out_shape=jax.ShapeDtypeStruct((P, L), jnp.float32),
        grid_spec=grid_spec,
        compiler_params=pltpu.CompilerParams(
            dimension_semantics=("arbitrary",),
            vmem_limit_bytes=64 * 1024 * 1024),
    )
    weights = (wih_s.astype(_MXU_DTYPE), bcat_s, whh_bd, w1_full, b1,
               w2_t.astype(_MXU_DTYPE), b2, w3_t.astype(_MXU_DTYPE), b3)

    outs = []
    for i in range(ntiles):
        tok = jnp.concatenate([sentence1[i * P:(i + 1) * P],
                               sentence2[i * P:(i + 1) * P]], axis=0)
        lens_i = jnp.concatenate(
            [lengths1[i * P:(i + 1) * P], lengths2[i * P:(i + 1) * P]],
            axis=0).reshape(Bt, 1).astype(jnp.int32)
        x_i = embedding.at[tok.T].get(mode="promise_in_bounds")
        outs.append(call(x_i, lens_i, *weights))
    return jnp.concatenate(outs, axis=0)


# speedup vs baseline: 1.3816x; 1.3816x over previous
import functools

import jax
import jax.numpy as jnp
from jax import lax
from jax.experimental import pallas as pl
from jax.experimental.pallas import tpu as pltpu

_MXU_DTYPE = jnp.bfloat16


def _fused_kernel(x_ref, lens_ref, wih_ref, bcat_ref, whhbd_ref,
                  w1_ref, b1_ref, w2_ref, b2_ref, w3_ref, b3_ref,
                  out_ref,
                  xbuf, xsem):
    T, Bt, E = x_ref.shape
    H = whhbd_ref.shape[0] // 2
    G4 = 4 * H

    CH = 4
    nch = (T + CH - 1) // CH

    def xcopy(c):
        lo, hi = c * CH, min(T, (c + 1) * CH)
        return pltpu.make_async_copy(x_ref.at[lo:hi], xbuf.at[lo:hi],
                                     xsem.at[c])

    order = []
    for k in range((nch + 1) // 2):
        order.append(k)
        if nch - 1 - k != k:
            order.append(nch - 1 - k)
    for c in order:
        xcopy(c).start()
    waited = set()

    bdt = whhbd_ref.dtype
    wih_f = wih_ref[:, :G4]
    wih_b = wih_ref[:, G4:]
    b_f = bcat_ref[:, :G4]
    b_b = bcat_ref[:, G4:]
    whh_f = whhbd_ref[:H]
    whh_b = whhbd_ref[H:]
    lens = lens_ref[...]

    def gate_math(g, c_prev):
        i = 0.5 * jnp.tanh(g[:, 0 * H:1 * H]) + 0.5
        f = 0.5 * jnp.tanh(g[:, 1 * H:2 * H]) + 0.5
        gg = jnp.tanh(g[:, 2 * H:3 * H])
        o = 0.5 * jnp.tanh(g[:, 3 * H:4 * H]) + 0.5
        c_new = f * c_prev + i * gg
        h_new = o * jnp.tanh(c_new)
        return h_new, c_new

    zeros_bf = jnp.zeros((Bt, H), bdt)
    zeros = jnp.zeros((Bt, H), jnp.float32)
    neg = jnp.full((Bt, H), -jnp.inf, jnp.float32)
    h_f, c_f, m_f, h_b, c_b, m_b = (
        zeros_bf, zeros, neg, zeros_bf, zeros, neg)

    for s in range(T):
        tb = T - 1 - s
        for c in (s // CH, tb // CH):
            if c not in waited:
                xcopy(c).wait()
                waited.add(c)
        gf = (jnp.dot(xbuf[s].astype(bdt), wih_f,
                      preferred_element_type=jnp.float32)
              + jnp.dot(h_f, whh_f, preferred_element_type=jnp.float32)
              + b_f)
        gb = (jnp.dot(xbuf[tb].astype(bdt), wih_b,
                      preferred_element_type=jnp.float32)
              + jnp.dot(h_b, whh_b, preferred_element_type=jnp.float32)
              + b_b)
        vf = s < lens
        vb = tb < lens
        hf_new, cf_new = gate_math(gf, c_f)
        hb_new, cb_new = gate_math(gb, c_b)
        m_f = jnp.maximum(m_f, jnp.where(vf, hf_new, 0.0))
        h_b = jnp.where(vb, hb_new.astype(bdt), h_b)
        c_b = jnp.where(vb, cb_new, c_b)
        m_b = jnp.maximum(m_b, jnp.where(vb, hb_new, 0.0))
        h_f, c_f = hf_new.astype(bdt), cf_new

    pooled = jnp.concatenate([m_f, m_b], axis=1)
    P = Bt // 2
    u = pooled[:P]
    v = pooled[P:]
    wdt = w1_ref.dtype
    feats = jnp.concatenate([u, v, jnp.abs(u - v), u * v],
                            axis=1).astype(wdt)
    h1 = (jnp.dot(feats, w1_ref[...], preferred_element_type=jnp.float32)
          + b1_ref[...])
    h2 = (jnp.dot(h1.astype(wdt), w2_ref[...],
                  preferred_element_type=jnp.float32) + b2_ref[...])
    out_ref[...] = (jnp.dot(h2.astype(wdt), w3_ref[...],
                            preferred_element_type=jnp.float32) + b3_ref[...])


def kernel(sentence1, lengths1, sentence2, lengths2, embedding,
           wih_cat_t, b_cat, whh_f_t, whh_b_t,
           w1_u, w1_v, w1_d, w1_p, b1, w2_t, b2, w3_t, b3):
    B, T = sentence1.shape
    E = embedding.shape[1]
    H = whh_f_t.shape[0]
    L = w3_t.shape[1]
    P = 128
    assert B % P == 0
    ntiles = B // P
    Bt = 2 * P

    gate_scale = jnp.concatenate(
        [jnp.full((1, H), 0.5, jnp.float32),
         jnp.full((1, H), 0.5, jnp.float32),
         jnp.ones((1, H), jnp.float32),
         jnp.full((1, H), 0.5, jnp.float32)], axis=1)
    gs2 = jnp.concatenate([gate_scale, gate_scale], axis=1)
    wih_s = wih_cat_t * gs2
    bcat_s = b_cat * gs2
    whh_bd = (jnp.concatenate([whh_f_t, whh_b_t], axis=0)
              * gate_scale).astype(_MXU_DTYPE)
    w1_full = jnp.concatenate([w1_u, w1_v, w1_d, w1_p],
                              axis=0).astype(_MXU_DTYPE)

    grid_spec = pltpu.PrefetchScalarGridSpec(
        num_scalar_prefetch=0,
        grid=(1,),
        in_specs=[
            pl.BlockSpec(memory_space=pltpu.MemorySpace.HBM),
            pl.BlockSpec((Bt, 1), lambda i: (0, 0)),
            pl.BlockSpec((E, 8 * H), lambda i: (0, 0)),
            pl.BlockSpec((1, 8 * H), lambda i: (0, 0)),
            pl.BlockSpec((2 * H, 4 * H), lambda i: (0, 0)),
            pl.BlockSpec((8 * H, w1_full.shape[1]), lambda i: (0, 0)),
            pl.BlockSpec((1, b1.shape[1]), lambda i: (0, 0)),
            pl.BlockSpec(w2_t.shape, lambda i: (0, 0)),
            pl.BlockSpec((1, b2.shape[1]), lambda i: (0, 0)),
            pl.BlockSpec(w3_t.shape, lambda i: (0, 0)),
            pl.BlockSpec((1, b3.shape[1]), lambda i: (0, 0)),
        ],
        out_specs=pl.BlockSpec((P, L), lambda i: (0, 0)),
        scratch_shapes=[pltpu.VMEM((T, Bt, E), jnp.float32),
                        pltpu.SemaphoreType.DMA(((T + 3) // 4,))],
    )
    call = pl.pallas_call(
        _fused_kernel,
        out_shape=jax.ShapeDtypeStruct((P, L), jnp.float32),
        grid_spec=grid_spec,
        compiler_params=pltpu.CompilerParams(
            dimension_semantics=("arbitrary",),
            vmem_limit_bytes=64 * 1024 * 1024),
    )
    weights = (wih_s.astype(_MXU_DTYPE), bcat_s, whh_bd, w1_full, b1,
               w2_t.astype(_MXU_DTYPE), b2, w3_t.astype(_MXU_DTYPE), b3)

    outs = []
    for i in range(ntiles):
        tok = jnp.concatenate([sentence1[i * P:(i + 1) * P],
                               sentence2[i * P:(i + 1) * P]], axis=0)
        lens_i = jnp.concatenate(
            [lengths1[i * P:(i + 1) * P], lengths2[i * P:(i + 1) * P]],
            axis=0).reshape(Bt, 1).astype(jnp.int32)
        x_i = embedding.at[tok.T].get(mode="promise_in_bounds")
        outs.append(call(x_i, lens_i, *weights))
    return jnp.concatenate(outs, axis=0)


# speedup vs baseline: 1.3994x; 1.0129x over previous
import functools

import jax
import jax.numpy as jnp
from jax import lax
from jax.experimental import pallas as pl
from jax.experimental.pallas import tpu as pltpu

_MXU_DTYPE = jnp.bfloat16


def _fused_kernel(x_ref, lens_ref, wih_ref, bcat_ref, whhbd_ref,
                  w1_ref, b1_ref, w2_ref, b2_ref, w3_ref, b3_ref,
                  out_ref):
    T, Bt, E = x_ref.shape
    H = whhbd_ref.shape[0] // 2
    G4 = 4 * H

    bdt = whhbd_ref.dtype
    wih_f = wih_ref[:, :G4]
    wih_b = wih_ref[:, G4:]
    b_f = bcat_ref[:, :G4]
    b_b = bcat_ref[:, G4:]
    whh_f = whhbd_ref[:H]
    whh_b = whhbd_ref[H:]
    lens = lens_ref[...]

    def gate_math(g, c_prev):
        i = 0.5 * jnp.tanh(g[:, 0 * H:1 * H]) + 0.5
        f = 0.5 * jnp.tanh(g[:, 1 * H:2 * H]) + 0.5
        gg = jnp.tanh(g[:, 2 * H:3 * H])
        o = 0.5 * jnp.tanh(g[:, 3 * H:4 * H]) + 0.5
        c_new = f * c_prev + i * gg
        h_new = o * jnp.tanh(c_new)
        return h_new, c_new

    def body(s, carry):
        h_f, c_f, m_f, h_b, c_b, m_b = carry
        tb = T - 1 - s
        gf = (jnp.dot(x_ref[s].astype(bdt), wih_f,
                      preferred_element_type=jnp.float32)
              + jnp.dot(h_f, whh_f, preferred_element_type=jnp.float32)
              + b_f)
        gb = (jnp.dot(x_ref[tb].astype(bdt), wih_b,
                      preferred_element_type=jnp.float32)
              + jnp.dot(h_b, whh_b, preferred_element_type=jnp.float32)
              + b_b)
        vf = s < lens
        vb = tb < lens
        hf_new, cf_new = gate_math(gf, c_f)
        hb_new, cb_new = gate_math(gb, c_b)
        m_f = jnp.maximum(m_f, jnp.where(vf, hf_new, 0.0))
        h_b = jnp.where(vb, hb_new.astype(bdt), h_b)
        c_b = jnp.where(vb, cb_new, c_b)
        m_b = jnp.maximum(m_b, jnp.where(vb, hb_new, 0.0))
        return hf_new.astype(bdt), cf_new, m_f, h_b, c_b, m_b

    zeros_bf = jnp.zeros((Bt, H), bdt)
    zeros = jnp.zeros((Bt, H), jnp.float32)
    neg = jnp.full((Bt, H), -jnp.inf, jnp.float32)
    init = (zeros_bf, zeros, neg, zeros_bf, zeros, neg)
    _, _, m_f, _, _, m_b = lax.fori_loop(0, T, body, init, unroll=T)

    pooled = jnp.concatenate([m_f, m_b], axis=1)
    P = Bt // 2
    u = pooled[:P]
    v = pooled[P:]
    wdt = w1_ref.dtype
    feats = jnp.concatenate([u, v, jnp.abs(u - v), u * v],
                            axis=1).astype(wdt)
    h1 = (jnp.dot(feats, w1_ref[...], preferred_element_type=jnp.float32)
          + b1_ref[...])
    h2 = (jnp.dot(h1.astype(wdt), w2_ref[...],
                  preferred_element_type=jnp.float32) + b2_ref[...])
    out_ref[...] = (jnp.dot(h2.astype(wdt), w3_ref[...],
                            preferred_element_type=jnp.float32) + b3_ref[...])


def kernel(sentence1, lengths1, sentence2, lengths2, embedding,
           wih_cat_t, b_cat, whh_f_t, whh_b_t,
           w1_u, w1_v, w1_d, w1_p, b1, w2_t, b2, w3_t, b3):
    B, T = sentence1.shape
    E = embedding.shape[1]
    H = whh_f_t.shape[0]
    L = w3_t.shape[1]
    P = 128
    assert B % P == 0
    ntiles = B // P
    Bt = 2 * P

    gate_scale = jnp.concatenate(
        [jnp.full((1, H), 0.5, jnp.float32),
         jnp.full((1, H), 0.5, jnp.float32),
         jnp.ones((1, H), jnp.float32),
         jnp.full((1, H), 0.5, jnp.float32)], axis=1)
    gs2 = jnp.concatenate([gate_scale, gate_scale], axis=1)
    wih_s = wih_cat_t * gs2
    bcat_s = b_cat * gs2
    whh_bd = (jnp.concatenate([whh_f_t, whh_b_t], axis=0)
              * gate_scale).astype(_MXU_DTYPE)
    w1_full = jnp.concatenate([w1_u, w1_v, w1_d, w1_p],
                              axis=0).astype(_MXU_DTYPE)

    grid_spec = pltpu.PrefetchScalarGridSpec(
        num_scalar_prefetch=0,
        grid=(1,),
        in_specs=[
            pl.BlockSpec((T, Bt, E), lambda i: (0, 0, 0)),
            pl.BlockSpec((Bt, 1), lambda i: (0, 0)),
            pl.BlockSpec((E, 8 * H), lambda i: (0, 0)),
            pl.BlockSpec((1, 8 * H), lambda i: (0, 0)),
            pl.BlockSpec((2 * H, 4 * H), lambda i: (0, 0)),
            pl.BlockSpec((8 * H, w1_full.shape[1]), lambda i: (0, 0)),
            pl.BlockSpec((1, b1.shape[1]), lambda i: (0, 0)),
            pl.BlockSpec(w2_t.shape, lambda i: (0, 0)),
            pl.BlockSpec((1, b2.shape[1]), lambda i: (0, 0)),
            pl.BlockSpec(w3_t.shape, lambda i: (0, 0)),
            pl.BlockSpec((1, b3.shape[1]), lambda i: (0, 0)),
        ],
        out_specs=pl.BlockSpec((P, L), lambda i: (0, 0)),
    )
    call = pl.pallas_call(
        _fused_kernel,
        out_shape=jax.ShapeDtypeStruct((P, L), jnp.float32),
        grid_spec=grid_spec,
        compiler_params=pltpu.CompilerParams(
            dimension_semantics=("arbitrary",),
            vmem_limit_bytes=64 * 1024 * 1024),
    )
    weights = (wih_s.astype(_MXU_DTYPE), bcat_s, whh_bd, w1_full, b1,
               w2_t.astype(_MXU_DTYPE), b2, w3_t.astype(_MXU_DTYPE), b3)

    outs = []
    for i in range(ntiles):
        tok = jnp.concatenate([sentence1[i * P:(i + 1) * P],
                               sentence2[i * P:(i + 1) * P]], axis=0)
        lens_i = jnp.concatenate(
            [lengths1[i * P:(i + 1) * P], lengths2[i * P:(i + 1) * P]],
            axis=0).reshape(Bt, 1).astype(jnp.int32)
        x_i = embedding.at[tok.T].get(mode="promise_in_bounds")
        outs.append(call(x_i, lens_i, *weights))
    return jnp.concatenate(outs, axis=0)
